# Initial kernel scaffold; baseline (speedup 1.0000x reference)
import jax
import jax.numpy as jnp
from jax.experimental import pallas as pl


def kernel(labels):
    raise NotImplementedError("write your pallas kernel here")


# speedup vs baseline: 1.8473x; 1.8473x over previous
import jax
import jax.numpy as jnp
from jax.experimental import pallas as pl

NC = 1000
VAL = 1001.0
ROWS_PER_BLOCK = 256


def _smooth_onehot_block(lab_ref, out_ref):
    lab = lab_ref[...]
    col = jax.lax.broadcasted_iota(jnp.int32, (lab.shape[0], NC), 1)
    out_ref[...] = jnp.where(lab == col, VAL, 1.0)


def kernel(labels):
    n = labels.shape[0]
    r = ROWS_PER_BLOCK
    lab2d = labels.astype(jnp.int32).reshape(n, 1)
    return pl.pallas_call(
        _smooth_onehot_block,
        grid=(n // r,),
        in_specs=[pl.BlockSpec((r, 1), lambda i: (i, 0))],
        out_specs=pl.BlockSpec((r, NC), lambda i: (i, 0)),
        out_shape=jax.ShapeDtypeStruct((n, NC), jnp.float32),
    )(lab2d)


# speedup vs baseline: 2.1323x; 1.1542x over previous
import jax
import jax.numpy as jnp
from jax.experimental import pallas as pl

NC = 1000
VAL = 1001.0
ROWS_PER_BLOCK = 512


def _smooth_onehot_block(lab_ref, out_ref):
    lab = lab_ref[...]
    col = jax.lax.broadcasted_iota(jnp.int32, (lab.shape[0], NC), 1)
    out_ref[...] = jnp.where(lab == col, VAL, 1.0)


def kernel(labels):
    n = labels.shape[0]
    r = ROWS_PER_BLOCK
    lab2d = labels.astype(jnp.int32).reshape(n, 1)
    return pl.pallas_call(
        _smooth_onehot_block,
        grid=(n // r,),
        in_specs=[pl.BlockSpec((r, 1), lambda i: (i, 0))],
        out_specs=pl.BlockSpec((r, NC), lambda i: (i, 0)),
        out_shape=jax.ShapeDtypeStruct((n, NC), jnp.float32),
    )(lab2d)


# speedup vs baseline: 2.3074x; 1.0821x over previous
import jax
import jax.numpy as jnp
from jax.experimental import pallas as pl

NC = 1000
VAL = 1001.0
ROWS_PER_BLOCK = 1024


def _smooth_onehot_block(lab_ref, out_ref):
    lab = lab_ref[...]
    col = jax.lax.broadcasted_iota(jnp.int32, (lab.shape[0], NC), 1)
    out_ref[...] = jnp.where(lab == col, VAL, 1.0)


def kernel(labels):
    n = labels.shape[0]
    r = ROWS_PER_BLOCK
    lab2d = labels.astype(jnp.int32).reshape(n, 1)
    return pl.pallas_call(
        _smooth_onehot_block,
        grid=(n // r,),
        in_specs=[pl.BlockSpec((r, 1), lambda i: (i, 0))],
        out_specs=pl.BlockSpec((r, NC), lambda i: (i, 0)),
        out_shape=jax.ShapeDtypeStruct((n, NC), jnp.float32),
    )(lab2d)


# speedup vs baseline: 2.3813x; 1.0320x over previous
import jax
import jax.numpy as jnp
from jax.experimental import pallas as pl

NC = 1000
VAL = 1001.0
ROWS_PER_BLOCK = 2048


def _smooth_onehot_block(lab_ref, out_ref):
    lab = lab_ref[...]
    col = jax.lax.broadcasted_iota(jnp.int32, (lab.shape[0], NC), 1)
    out_ref[...] = jnp.where(lab == col, VAL, 1.0)


def kernel(labels):
    n = labels.shape[0]
    r = ROWS_PER_BLOCK
    lab2d = labels.astype(jnp.int32).reshape(n, 1)
    return pl.pallas_call(
        _smooth_onehot_block,
        grid=(n // r,),
        in_specs=[pl.BlockSpec((r, 1), lambda i: (i, 0))],
        out_specs=pl.BlockSpec((r, NC), lambda i: (i, 0)),
        out_shape=jax.ShapeDtypeStruct((n, NC), jnp.float32),
    )(lab2d)
